# baseline (speedup 1.0000x reference)
import functools

import numpy as np
import jax
import jax.numpy as jnp
from jax import lax
from jax.experimental import pallas as pl
from jax.experimental.pallas import tpu as pltpu
from jax.experimental.pallas import tpu_sc as plsc

D = 128
K = 64
NNEG = 20
LANES = 16
NC, NS = 2, 16
NW = NC * NS
CB = 128
SB = 4
NGRP = CB // SB

_GAMMA = float(0.01 * 10.0 ** (-1 * np.log10(0.01) / (80 * 10 * 100000 * 10)))


def _ds16(c):
    return pl.ds(c * LANES, LANES)


@functools.cache
def _sc_gather(B):
    BPW = B // NW
    NCHUNK = BPW // CB
    mesh = plsc.VectorSubcoreMesh(core_axis_name="c", subcore_axis_name="s")

    def body(emb_hbm, uidx_hbm, vidx_hbm, nidx_hbm,
             u_out, pos_out, neg_out,
             uidx_v, vidx_v, nidx_v, urows_v, vrows_v, nsum_v,
             pdots_v, ndots_v, pos_v, neg_v, nbuf0, nbuf1,
             sem_uv, sem_n0, sem_n1):
        wid = lax.axis_index("s") * NC + lax.axis_index("c")
        nbufs = (nbuf0, nbuf1)
        nsems = (sem_n0, sem_n1)
        NBUF = len(nbufs)

        def chunk_body(ci, carry):
            base = wid * BPW + ci * CB
            pltpu.sync_copy(uidx_hbm.at[pl.ds(base, CB)], uidx_v)
            pltpu.sync_copy(vidx_hbm.at[pl.ds(base, CB)], vidx_v)
            pltpu.sync_copy(nidx_hbm.at[pl.ds(base * NNEG, CB * NNEG)], nidx_v)
            cp_u = pltpu.async_copy(emb_hbm.at[uidx_v], urows_v, sem_uv)
            cp_v = pltpu.async_copy(emb_hbm.at[vidx_v], vrows_v, sem_uv)
            for s in range(NBUF):
                pltpu.async_copy(
                    emb_hbm.at[nidx_v.at[pl.ds(s * (SB * NNEG), SB * NNEG)]],
                    nbufs[s], nsems[s])

            def g_body(g2, carry2):
                for s in range(NBUF):
                    g = g2 * NBUF + s
                    buf = nbufs[s]
                    pltpu.make_async_copy(
                        emb_hbm.at[nidx_v.at[pl.ds(g * (SB * NNEG), SB * NNEG)]],
                        buf, nsems[s]).wait()
                    for bi in range(SB):
                        r0 = bi * NNEG
                        acc = [buf[r0, _ds16(c)] for c in range(D // LANES)]
                        for r in range(1, NNEG):
                            for c in range(D // LANES):
                                acc[c] = acc[c] + buf[r0 + r, _ds16(c)]
                        for c in range(D // LANES):
                            nsum_v[pl.ds((g * SB + bi) * D + c * LANES, LANES)] = acc[c]
                    ng = g + NBUF

                    @pl.when(ng < NGRP)
                    def _():
                        pltpu.async_copy(
                            emb_hbm.at[nidx_v.at[pl.ds(ng * (SB * NNEG), SB * NNEG)]],
                            buf, nsems[s])
                return carry2

            lax.fori_loop(0, NGRP // NBUF, g_body, 0)
            cp_u.wait()
            cp_v.wait()

            def b_body(b, carry2):
                uv0 = urows_v[b, _ds16(0)]
                pacc = uv0 * vrows_v[b, _ds16(0)]
                nacc = uv0 * nsum_v[pl.ds(b * D, LANES)]
                for c in range(1, D // LANES):
                    uvc = urows_v[b, _ds16(c)]
                    pacc = pacc + uvc * vrows_v[b, _ds16(c)]
                    nacc = nacc + uvc * nsum_v[pl.ds(b * D + c * LANES, LANES)]
                pdots_v[pl.ds(b * LANES, LANES)] = pacc
                ndots_v[pl.ds(b * LANES, LANES)] = nacc
                return carry2

            lax.fori_loop(0, CB, b_body, 0)
            lanes = lax.iota(jnp.int32, LANES)
            for g in range(CB // LANES):
                pvec = jnp.zeros((LANES,), jnp.float32)
                nvec = jnp.zeros((LANES,), jnp.float32)
                for j in range(LANES):
                    b = g * LANES + j
                    ps = jnp.sum(pdots_v[pl.ds(b * LANES, LANES)])
                    nsc = jnp.sum(ndots_v[pl.ds(b * LANES, LANES)])
                    pvec = jnp.where(lanes == j, ps, pvec)
                    nvec = jnp.where(lanes == j, nsc, nvec)
                pos_v[_ds16(g)] = pvec
                neg_v[_ds16(g)] = -nvec

            pltpu.sync_copy(urows_v, u_out.at[pl.ds(base, CB)])
            pltpu.sync_copy(pos_v, pos_out.at[pl.ds(base, CB)])
            pltpu.sync_copy(neg_v, neg_out.at[pl.ds(base, CB)])
            return carry

        lax.fori_loop(0, NCHUNK, chunk_body, 0)

    return pl.kernel(
        body,
        out_type=(
            jax.ShapeDtypeStruct((B, D), jnp.float32),
            jax.ShapeDtypeStruct((B,), jnp.float32),
            jax.ShapeDtypeStruct((B,), jnp.float32),
        ),
        mesh=mesh,
        compiler_params=pltpu.CompilerParams(needs_layout_passes=False),
        cost_estimate=pl.CostEstimate(
            flops=B * NNEG * D * 2,
            bytes_accessed=(B * (NNEG + 2) * D + B * D) * 4,
            transcendentals=0),
        scratch_types=(
            pltpu.VMEM((CB,), jnp.int32),
            pltpu.VMEM((CB,), jnp.int32),
            pltpu.VMEM((CB * NNEG,), jnp.int32),
            pltpu.VMEM((CB, D), jnp.float32),
            pltpu.VMEM((CB, D), jnp.float32),
            pltpu.VMEM((CB * D,), jnp.float32),
            pltpu.VMEM((CB * LANES,), jnp.float32),
            pltpu.VMEM((CB * LANES,), jnp.float32),
            pltpu.VMEM((CB,), jnp.float32),
            pltpu.VMEM((CB,), jnp.float32),
            pltpu.VMEM((SB * NNEG, D), jnp.float32),
            pltpu.VMEM((SB * NNEG, D), jnp.float32),
            pltpu.SemaphoreType.DMA,
            pltpu.SemaphoreType.DMA,
            pltpu.SemaphoreType.DMA,
        ),
    )


def _halve_reduce(x):
    while x.shape[0] > 1:
        h = x.shape[0] // 2
        x = x[:h, :] + x[h:, :]
    return x


def _tc_body(ut_ref, pos_ref, neg_ref, comt_ref, loss_ref, cc_ref, acc_ref):
    i = pl.program_id(0)
    blk = ut_ref.shape[1]
    ut = ut_ref[...]
    lsum = jnp.sum(jax.nn.log_sigmoid(pos_ref[...])
                   + jax.nn.log_sigmoid(neg_ref[...]))
    diff = ut - comt_ref[:, 0:1]
    best = jnp.sqrt(_halve_reduce(diff * diff))
    besti = jnp.zeros((1, blk), jnp.int32)
    for k in range(1, K):
        diff = ut - comt_ref[:, k:k + 1]
        dk = jnp.sqrt(_halve_reduce(diff * diff))
        better = dk < best
        besti = jnp.where(better, k, besti)
        best = jnp.where(better, dk, best)
    cc_ref[...] = besti[0]
    min2 = jnp.sum(best * best)

    @pl.when(i == 0)
    def _():
        acc_ref[0] = 0.0
        acc_ref[1] = 0.0

    acc_ref[0] = acc_ref[0] + lsum
    acc_ref[1] = acc_ref[1] + min2

    @pl.when(i == pl.num_programs(0) - 1)
    def _():
        loss_ref[...] = jnp.stack([acc_ref[0], acc_ref[1]]).reshape(1, 2)


def _tc_stage(ut, pos, neg, comt):
    BLK = 2048
    BS = ut.shape[1]
    return pl.pallas_call(
        _tc_body,
        grid=(BS // BLK,),
        in_specs=[
            pl.BlockSpec((D, BLK), lambda i: (0, i)),
            pl.BlockSpec((BLK,), lambda i: (i,)),
            pl.BlockSpec((BLK,), lambda i: (i,)),
            pl.BlockSpec((D, K), lambda i: (0, 0)),
        ],
        out_specs=(
            pl.BlockSpec((1, 2), lambda i: (0, 0)),
            pl.BlockSpec((BLK,), lambda i: (i,)),
        ),
        out_shape=(
            jax.ShapeDtypeStruct((1, 2), jnp.float32),
            jax.ShapeDtypeStruct((BS,), jnp.int32),
        ),
        scratch_shapes=[pltpu.SMEM((2,), jnp.float32)],
    )(ut, pos, neg, comt)


_NST = 2


def kernel(u_node, v_node, negative_nodes, nb_labels, emb_u, emb_com):
    B = u_node.shape[0]
    BS = B // _NST
    uidx = u_node.reshape(B).astype(jnp.int32)
    vidx = v_node.reshape(B).astype(jnp.int32)
    nidx = negative_nodes.reshape(B, NNEG).astype(jnp.int32)
    comt = emb_com.T
    sc_outs = []
    for s in range(_NST):
        lo = s * BS
        sc_outs.append(_sc_gather(BS)(
            emb_u, uidx[lo:lo + BS], vidx[lo:lo + BS],
            nidx[lo:lo + BS].reshape(BS * NNEG)))
    parts, ccs = [], []
    for u_rows, pos, neg in sc_outs:
        part, cc_s = _tc_stage(u_rows.T, pos, neg, comt)
        parts.append(part)
        ccs.append(cc_s)
    tot = parts[0]
    for p in parts[1:]:
        tot = tot + p
    final = -(tot[0, 0] / B) + _GAMMA * (tot[0, 1] / B)
    return final, jnp.concatenate(ccs)


# speedup vs baseline: 1.0124x; 1.0124x over previous
import functools

import numpy as np
import jax
import jax.numpy as jnp
from jax import lax
from jax.experimental import pallas as pl
from jax.experimental.pallas import tpu as pltpu
from jax.experimental.pallas import tpu_sc as plsc

D = 128
K = 64
NNEG = 20
LANES = 16
NC, NS = 2, 16
NW = NC * NS
CB = 128
SB = 4
NGRP = CB // SB

_GAMMA = float(0.01 * 10.0 ** (-1 * np.log10(0.01) / (80 * 10 * 100000 * 10)))


def _ds16(c):
    return pl.ds(c * LANES, LANES)


@functools.cache
def _sc_gather(B):
    BPW = B // NW
    NCHUNK = BPW // CB
    mesh = plsc.VectorSubcoreMesh(core_axis_name="c", subcore_axis_name="s")

    def body(emb_hbm, uidx_hbm, vidx_hbm, nidx_hbm,
             u_out, pos_out, neg_out,
             uidx_v, vidx_v, nidx_v, urows_v, vrows_v, nsum_v,
             pdots_v, ndots_v, pos_v, neg_v, nbuf0, nbuf1,
             sem_uv, sem_n0, sem_n1):
        wid = lax.axis_index("s") * NC + lax.axis_index("c")
        nbufs = (nbuf0, nbuf1)
        nsems = (sem_n0, sem_n1)
        NBUF = len(nbufs)

        def chunk_body(ci, carry):
            base = wid * BPW + ci * CB
            pltpu.sync_copy(uidx_hbm.at[pl.ds(base, CB)], uidx_v)
            pltpu.sync_copy(vidx_hbm.at[pl.ds(base, CB)], vidx_v)
            pltpu.sync_copy(nidx_hbm.at[pl.ds(base * NNEG, CB * NNEG)], nidx_v)
            cp_u = pltpu.async_copy(emb_hbm.at[uidx_v], urows_v, sem_uv)
            cp_v = pltpu.async_copy(emb_hbm.at[vidx_v], vrows_v, sem_uv)
            for s in range(NBUF):
                pltpu.async_copy(
                    emb_hbm.at[nidx_v.at[pl.ds(s * (SB * NNEG), SB * NNEG)]],
                    nbufs[s], nsems[s])

            def g_body(g2, carry2):
                for s in range(NBUF):
                    g = g2 * NBUF + s
                    buf = nbufs[s]
                    pltpu.make_async_copy(
                        emb_hbm.at[nidx_v.at[pl.ds(g * (SB * NNEG), SB * NNEG)]],
                        buf, nsems[s]).wait()
                    for bi in range(SB):
                        r0 = bi * NNEG
                        acc = [buf[r0, _ds16(c)] for c in range(D // LANES)]
                        for r in range(1, NNEG):
                            for c in range(D // LANES):
                                acc[c] = acc[c] + buf[r0 + r, _ds16(c)]
                        for c in range(D // LANES):
                            nsum_v[pl.ds((g * SB + bi) * D + c * LANES, LANES)] = acc[c]
                    ng = g + NBUF

                    @pl.when(ng < NGRP)
                    def _():
                        pltpu.async_copy(
                            emb_hbm.at[nidx_v.at[pl.ds(ng * (SB * NNEG), SB * NNEG)]],
                            buf, nsems[s])
                return carry2

            lax.fori_loop(0, NGRP // NBUF, g_body, 0)
            cp_u.wait()
            cp_v.wait()

            def b_body(b, carry2):
                uv0 = urows_v[b, _ds16(0)]
                pacc = uv0 * vrows_v[b, _ds16(0)]
                nacc = uv0 * nsum_v[pl.ds(b * D, LANES)]
                for c in range(1, D // LANES):
                    uvc = urows_v[b, _ds16(c)]
                    pacc = pacc + uvc * vrows_v[b, _ds16(c)]
                    nacc = nacc + uvc * nsum_v[pl.ds(b * D + c * LANES, LANES)]
                pdots_v[pl.ds(b * LANES, LANES)] = pacc
                ndots_v[pl.ds(b * LANES, LANES)] = nacc
                return carry2

            lax.fori_loop(0, CB, b_body, 0)
            lanes = lax.iota(jnp.int32, LANES)
            for g in range(CB // LANES):
                pvec = jnp.zeros((LANES,), jnp.float32)
                nvec = jnp.zeros((LANES,), jnp.float32)
                for j in range(LANES):
                    b = g * LANES + j
                    ps = jnp.sum(pdots_v[pl.ds(b * LANES, LANES)])
                    nsc = jnp.sum(ndots_v[pl.ds(b * LANES, LANES)])
                    pvec = jnp.where(lanes == j, ps, pvec)
                    nvec = jnp.where(lanes == j, nsc, nvec)
                pos_v[_ds16(g)] = pvec
                neg_v[_ds16(g)] = -nvec

            pltpu.sync_copy(urows_v, u_out.at[pl.ds(base, CB)])
            pltpu.sync_copy(pos_v, pos_out.at[pl.ds(base, CB)])
            pltpu.sync_copy(neg_v, neg_out.at[pl.ds(base, CB)])
            return carry

        lax.fori_loop(0, NCHUNK, chunk_body, 0)

    return pl.kernel(
        body,
        out_type=(
            jax.ShapeDtypeStruct((B, D), jnp.float32),
            jax.ShapeDtypeStruct((B,), jnp.float32),
            jax.ShapeDtypeStruct((B,), jnp.float32),
        ),
        mesh=mesh,
        compiler_params=pltpu.CompilerParams(needs_layout_passes=False),
        cost_estimate=pl.CostEstimate(
            flops=B * NNEG * D * 2,
            bytes_accessed=(B * (NNEG + 2) * D + B * D) * 4,
            transcendentals=0),
        scratch_types=(
            pltpu.VMEM((CB,), jnp.int32),
            pltpu.VMEM((CB,), jnp.int32),
            pltpu.VMEM((CB * NNEG,), jnp.int32),
            pltpu.VMEM((CB, D), jnp.float32),
            pltpu.VMEM((CB, D), jnp.float32),
            pltpu.VMEM((CB * D,), jnp.float32),
            pltpu.VMEM((CB * LANES,), jnp.float32),
            pltpu.VMEM((CB * LANES,), jnp.float32),
            pltpu.VMEM((CB,), jnp.float32),
            pltpu.VMEM((CB,), jnp.float32),
            pltpu.VMEM((SB * NNEG, D), jnp.float32),
            pltpu.VMEM((SB * NNEG, D), jnp.float32),
            pltpu.SemaphoreType.DMA,
            pltpu.SemaphoreType.DMA,
            pltpu.SemaphoreType.DMA,
        ),
    )


def _halve_reduce(x):
    while x.shape[0] > 1:
        h = x.shape[0] // 2
        x = x[:h, :] + x[h:, :]
    return x


def _tc_body(ut_ref, pos_ref, neg_ref, comt_ref, loss_ref, cc_ref, acc_ref):
    i = pl.program_id(0)
    blk = ut_ref.shape[1]
    ut = ut_ref[...]
    lsum = jnp.sum(jax.nn.log_sigmoid(pos_ref[...])
                   + jax.nn.log_sigmoid(neg_ref[...]))
    diff = ut - comt_ref[:, 0:1]
    best = jnp.sqrt(_halve_reduce(diff * diff))
    besti = jnp.zeros((1, blk), jnp.int32)
    for k in range(1, K):
        diff = ut - comt_ref[:, k:k + 1]
        dk = jnp.sqrt(_halve_reduce(diff * diff))
        better = dk < best
        besti = jnp.where(better, k, besti)
        best = jnp.where(better, dk, best)
    cc_ref[...] = besti[0]
    min2 = jnp.sum(best * best)

    @pl.when(i == 0)
    def _():
        acc_ref[0] = 0.0
        acc_ref[1] = 0.0

    acc_ref[0] = acc_ref[0] + lsum
    acc_ref[1] = acc_ref[1] + min2

    @pl.when(i == pl.num_programs(0) - 1)
    def _():
        loss_ref[...] = jnp.stack([acc_ref[0], acc_ref[1]]).reshape(1, 2)


def _tc_stage(ut, pos, neg, comt):
    BLK = 2048
    BS = ut.shape[1]
    return pl.pallas_call(
        _tc_body,
        grid=(BS // BLK,),
        in_specs=[
            pl.BlockSpec((D, BLK), lambda i: (0, i)),
            pl.BlockSpec((BLK,), lambda i: (i,)),
            pl.BlockSpec((BLK,), lambda i: (i,)),
            pl.BlockSpec((D, K), lambda i: (0, 0)),
        ],
        out_specs=(
            pl.BlockSpec((1, 2), lambda i: (0, 0)),
            pl.BlockSpec((BLK,), lambda i: (i,)),
        ),
        out_shape=(
            jax.ShapeDtypeStruct((1, 2), jnp.float32),
            jax.ShapeDtypeStruct((BS,), jnp.int32),
        ),
        scratch_shapes=[pltpu.SMEM((2,), jnp.float32)],
    )(ut, pos, neg, comt)


_NST = 1


def kernel(u_node, v_node, negative_nodes, nb_labels, emb_u, emb_com):
    B = u_node.shape[0]
    BS = B // _NST
    uidx = u_node.reshape(B).astype(jnp.int32)
    vidx = v_node.reshape(B).astype(jnp.int32)
    nidx = negative_nodes.reshape(B, NNEG).astype(jnp.int32)
    comt = emb_com.T
    sc_outs = []
    for s in range(_NST):
        lo = s * BS
        sc_outs.append(_sc_gather(BS)(
            emb_u, uidx[lo:lo + BS], vidx[lo:lo + BS],
            nidx[lo:lo + BS].reshape(BS * NNEG)))
    parts, ccs = [], []
    for u_rows, pos, neg in sc_outs:
        part, cc_s = _tc_stage(u_rows.T, pos, neg, comt)
        parts.append(part)
        ccs.append(cc_s)
    tot = parts[0]
    for p in parts[1:]:
        tot = tot + p
    final = -(tot[0, 0] / B) + _GAMMA * (tot[0, 1] / B)
    return final, jnp.concatenate(ccs)


# speedup vs baseline: 1.0546x; 1.0417x over previous
import functools

import numpy as np
import jax
import jax.numpy as jnp
from jax import lax
from jax.experimental import pallas as pl
from jax.experimental.pallas import tpu as pltpu
from jax.experimental.pallas import tpu_sc as plsc

D = 128
K = 64
NNEG = 20
LANES = 16
NC, NS = 2, 16
NW = NC * NS
CB = 128
SB = 4
NGRP = CB // SB

_GAMMA = float(0.01 * 10.0 ** (-1 * np.log10(0.01) / (80 * 10 * 100000 * 10)))


def _ds16(c):
    return pl.ds(c * LANES, LANES)


@functools.cache
def _sc_gather(B):
    BPW = B // NW
    NCHUNK = BPW // CB
    mesh = plsc.VectorSubcoreMesh(core_axis_name="c", subcore_axis_name="s")

    def body(emb_hbm, uidx_hbm, vidx_hbm, nidx_hbm,
             u_out, pos_out, neg_out,
             uidx_v, vidx_v, nidx_v, urows_v, vrows_v, nsum_v,
             pdots_v, ndots_v, pos_v, neg_v, nbuf0, nbuf1,
             sem_uv, sem_n0, sem_n1):
        wid = lax.axis_index("s") * NC + lax.axis_index("c")
        nbufs = (nbuf0, nbuf1)
        nsems = (sem_n0, sem_n1)
        NBUF = len(nbufs)

        def chunk_body(ci, carry):
            base = wid * BPW + ci * CB
            pltpu.sync_copy(uidx_hbm.at[pl.ds(base, CB)], uidx_v)
            pltpu.sync_copy(vidx_hbm.at[pl.ds(base, CB)], vidx_v)
            pltpu.sync_copy(nidx_hbm.at[pl.ds(base * NNEG, CB * NNEG)], nidx_v)
            cp_u = pltpu.async_copy(emb_hbm.at[uidx_v], urows_v, sem_uv)
            cp_v = pltpu.async_copy(emb_hbm.at[vidx_v], vrows_v, sem_uv)
            for s in range(NBUF):
                pltpu.async_copy(
                    emb_hbm.at[nidx_v.at[pl.ds(s * (SB * NNEG), SB * NNEG)]],
                    nbufs[s], nsems[s])

            def g_body(g2, carry2):
                for s in range(NBUF):
                    g = g2 * NBUF + s
                    buf = nbufs[s]
                    pltpu.make_async_copy(
                        emb_hbm.at[nidx_v.at[pl.ds(g * (SB * NNEG), SB * NNEG)]],
                        buf, nsems[s]).wait()
                    for bi in range(SB):
                        r0 = bi * NNEG
                        acc = [buf[r0, _ds16(c)] for c in range(D // LANES)]
                        for r in range(1, NNEG):
                            for c in range(D // LANES):
                                acc[c] = acc[c] + buf[r0 + r, _ds16(c)]
                        for c in range(D // LANES):
                            nsum_v[pl.ds((g * SB + bi) * D + c * LANES, LANES)] = acc[c]
                    ng = g + NBUF

                    @pl.when(ng < NGRP)
                    def _():
                        pltpu.async_copy(
                            emb_hbm.at[nidx_v.at[pl.ds(ng * (SB * NNEG), SB * NNEG)]],
                            buf, nsems[s])
                return carry2

            lax.fori_loop(0, NGRP // NBUF, g_body, 0)
            cp_u.wait()
            cp_v.wait()

            def b_body(b, carry2):
                uv0 = urows_v[b, _ds16(0)]
                pacc = uv0 * vrows_v[b, _ds16(0)]
                nacc = uv0 * nsum_v[pl.ds(b * D, LANES)]
                for c in range(1, D // LANES):
                    uvc = urows_v[b, _ds16(c)]
                    pacc = pacc + uvc * vrows_v[b, _ds16(c)]
                    nacc = nacc + uvc * nsum_v[pl.ds(b * D + c * LANES, LANES)]
                pdots_v[pl.ds(b * LANES, LANES)] = pacc
                ndots_v[pl.ds(b * LANES, LANES)] = nacc
                return carry2

            lax.fori_loop(0, CB, b_body, 0)
            lanes = lax.iota(jnp.int32, LANES)
            for g in range(CB // LANES):
                pvec = jnp.zeros((LANES,), jnp.float32)
                nvec = jnp.zeros((LANES,), jnp.float32)
                for j in range(LANES):
                    b = g * LANES + j
                    ps = jnp.sum(pdots_v[pl.ds(b * LANES, LANES)])
                    nsc = jnp.sum(ndots_v[pl.ds(b * LANES, LANES)])
                    pvec = jnp.where(lanes == j, ps, pvec)
                    nvec = jnp.where(lanes == j, nsc, nvec)
                pos_v[_ds16(g)] = pvec
                neg_v[_ds16(g)] = -nvec

            pltpu.sync_copy(urows_v, u_out.at[pl.ds(base, CB)])
            pltpu.sync_copy(pos_v, pos_out.at[pl.ds(base, CB)])
            pltpu.sync_copy(neg_v, neg_out.at[pl.ds(base, CB)])
            return carry

        lax.fori_loop(0, NCHUNK, chunk_body, 0)

    return pl.kernel(
        body,
        out_type=(
            jax.ShapeDtypeStruct((B, D), jnp.float32),
            jax.ShapeDtypeStruct((B,), jnp.float32),
            jax.ShapeDtypeStruct((B,), jnp.float32),
        ),
        mesh=mesh,
        compiler_params=pltpu.CompilerParams(needs_layout_passes=False),
        cost_estimate=pl.CostEstimate(
            flops=B * NNEG * D * 2,
            bytes_accessed=(B * (NNEG + 2) * D + B * D) * 4,
            transcendentals=0),
        scratch_types=(
            pltpu.VMEM((CB,), jnp.int32),
            pltpu.VMEM((CB,), jnp.int32),
            pltpu.VMEM((CB * NNEG,), jnp.int32),
            pltpu.VMEM((CB, D), jnp.float32),
            pltpu.VMEM((CB, D), jnp.float32),
            pltpu.VMEM((CB * D,), jnp.float32),
            pltpu.VMEM((CB * LANES,), jnp.float32),
            pltpu.VMEM((CB * LANES,), jnp.float32),
            pltpu.VMEM((CB,), jnp.float32),
            pltpu.VMEM((CB,), jnp.float32),
            pltpu.VMEM((SB * NNEG, D), jnp.float32),
            pltpu.VMEM((SB * NNEG, D), jnp.float32),
            pltpu.SemaphoreType.DMA,
            pltpu.SemaphoreType.DMA,
            pltpu.SemaphoreType.DMA,
        ),
    )


def _halve_reduce(x):
    while x.shape[0] > 1:
        h = x.shape[0] // 2
        x = x[:h, :] + x[h:, :]
    return x


def _tc_body(ut_ref, pos_ref, neg_ref, comt_ref, loss_ref, cc_ref, acc_ref):
    i = pl.program_id(0)
    blk = ut_ref.shape[1]
    ut = ut_ref[...]
    lsum = jnp.sum(jax.nn.log_sigmoid(pos_ref[...])
                   + jax.nn.log_sigmoid(neg_ref[...]))
    diff = ut - comt_ref[:, 0:1]
    best = jnp.sqrt(_halve_reduce(diff * diff))
    besti = jnp.zeros((1, blk), jnp.int32)
    for k in range(1, K):
        diff = ut - comt_ref[:, k:k + 1]
        dk = jnp.sqrt(_halve_reduce(diff * diff))
        better = dk < best
        besti = jnp.where(better, k, besti)
        best = jnp.where(better, dk, best)
    cc_ref[...] = besti[0]
    min2 = jnp.sum(best * best)

    @pl.when(i == 0)
    def _():
        acc_ref[0] = 0.0
        acc_ref[1] = 0.0

    acc_ref[0] = acc_ref[0] + lsum
    acc_ref[1] = acc_ref[1] + min2

    @pl.when(i == pl.num_programs(0) - 1)
    def _():
        loss_ref[...] = jnp.stack([acc_ref[0], acc_ref[1]]).reshape(1, 2)


def _tc_stage(ut, pos, neg, comt):
    BLK = 1024
    BS = ut.shape[1]
    return pl.pallas_call(
        _tc_body,
        grid=(BS // BLK,),
        in_specs=[
            pl.BlockSpec((D, BLK), lambda i: (0, i)),
            pl.BlockSpec((BLK,), lambda i: (i,)),
            pl.BlockSpec((BLK,), lambda i: (i,)),
            pl.BlockSpec((D, K), lambda i: (0, 0)),
        ],
        out_specs=(
            pl.BlockSpec((1, 2), lambda i: (0, 0)),
            pl.BlockSpec((BLK,), lambda i: (i,)),
        ),
        out_shape=(
            jax.ShapeDtypeStruct((1, 2), jnp.float32),
            jax.ShapeDtypeStruct((BS,), jnp.int32),
        ),
        scratch_shapes=[pltpu.SMEM((2,), jnp.float32)],
    )(ut, pos, neg, comt)


_NST = 1


def kernel(u_node, v_node, negative_nodes, nb_labels, emb_u, emb_com):
    B = u_node.shape[0]
    BS = B // _NST
    uidx = u_node.reshape(B).astype(jnp.int32)
    vidx = v_node.reshape(B).astype(jnp.int32)
    nidx = negative_nodes.reshape(B, NNEG).astype(jnp.int32)
    comt = emb_com.T
    sc_outs = []
    for s in range(_NST):
        lo = s * BS
        sc_outs.append(_sc_gather(BS)(
            emb_u, uidx[lo:lo + BS], vidx[lo:lo + BS],
            nidx[lo:lo + BS].reshape(BS * NNEG)))
    parts, ccs = [], []
    for u_rows, pos, neg in sc_outs:
        part, cc_s = _tc_stage(u_rows.T, pos, neg, comt)
        parts.append(part)
        ccs.append(cc_s)
    tot = parts[0]
    for p in parts[1:]:
        tot = tot + p
    final = -(tot[0, 0] / B) + _GAMMA * (tot[0, 1] / B)
    return final, jnp.concatenate(ccs)
